# Initial kernel scaffold; baseline (speedup 1.0000x reference)
import jax
import jax.numpy as jnp
from jax.experimental import pallas as pl


def kernel(z, embedding):
    raise NotImplementedError("write your pallas kernel here")


# speedup vs baseline: 1.1014x; 1.1014x over previous
import jax
import jax.numpy as jnp
from jax.experimental import pallas as pl
from jax.experimental.pallas import tpu as pltpu

_N_E = 8192
_E = 32
_BETA = 0.25
_NTOK = 8192
_T = 256
_NT = _NTOK // _T


def _vq_body(zf_ref, emb_ref, oh_ref, zq_ref, idx_ref, loss_ref, ppl_ref,
             acc_ref, cnt_ref):
    i = pl.program_id(0)
    zf = zf_ref[...]
    emb = emb_ref[...]
    mm = jnp.dot(zf, emb.T, preferred_element_type=jnp.float32)
    zf_sq = jnp.sum(zf * zf, axis=1, keepdims=True)
    emb_sq = jnp.sum(emb * emb, axis=1)[None, :]
    d = (zf_sq + emb_sq) - 2.0 * mm
    dmin = jnp.min(d, axis=1, keepdims=True)
    iota = jax.lax.broadcasted_iota(jnp.int32, (_T, _N_E), 1)
    cand = jnp.where(d == dmin, iota, _N_E)
    idx = jnp.min(cand, axis=1)
    oh = (iota == idx[:, None]).astype(jnp.float32)
    oh_ref[...] = oh
    zq = jnp.dot(oh, emb, preferred_element_type=jnp.float32)
    zq_ref[...] = zq
    idx_ref[...] = idx[None, None, :]

    diff = zq - zf
    part = jnp.sum(diff * diff, axis=(0, 1), keepdims=True)
    cnt = jnp.sum(oh, axis=0)[None, :]

    @pl.when(i == 0)
    def _init():
        acc_ref[...] = jnp.zeros((1, 1), jnp.float32)
        cnt_ref[...] = jnp.zeros((1, _N_E), jnp.float32)

    acc_ref[...] = acc_ref[...] + part
    cnt_ref[...] = cnt_ref[...] + cnt

    @pl.when(i == _NT - 1)
    def _fin():
        numel = float(_NTOK * _E)
        m = acc_ref[...] * (1.0 / numel)
        loss_ref[...] = m + _BETA * m
        e_mean = cnt_ref[...] * (1.0 / _NTOK)
        ent = jnp.sum(e_mean * jnp.log(e_mean + 1e-10), axis=(0, 1),
                      keepdims=True)
        ppl_ref[...] = jnp.exp(-ent)


def _vq_call(zf, embedding, interpret=False):
    return pl.pallas_call(
        _vq_body,
        grid=(_NT,),
        in_specs=[
            pl.BlockSpec((_T, _E), lambda i: (i, 0)),
            pl.BlockSpec((_N_E, _E), lambda i: (0, 0)),
        ],
        out_specs=[
            pl.BlockSpec((_T, _N_E), lambda i: (i, 0)),
            pl.BlockSpec((_T, _E), lambda i: (i, 0)),
            pl.BlockSpec((1, 1, _T), lambda i: (i, 0, 0)),
            pl.BlockSpec((1, 1), lambda i: (0, 0)),
            pl.BlockSpec((1, 1), lambda i: (0, 0)),
        ],
        out_shape=[
            jax.ShapeDtypeStruct((_NTOK, _N_E), jnp.float32),
            jax.ShapeDtypeStruct((_NTOK, _E), jnp.float32),
            jax.ShapeDtypeStruct((_NT, 1, _T), jnp.int32),
            jax.ShapeDtypeStruct((1, 1), jnp.float32),
            jax.ShapeDtypeStruct((1, 1), jnp.float32),
        ],
        scratch_shapes=[
            pltpu.VMEM((1, 1), jnp.float32),
            pltpu.VMEM((1, _N_E), jnp.float32),
        ],
        interpret=interpret,
    )(zf, embedding)


def kernel(z, embedding):
    zp = jnp.transpose(z, (0, 2, 3, 4, 1))
    zf = zp.reshape(_NTOK, _E)
    oh, zq, idx3, loss, ppl = _vq_call(zf, embedding)
    z_q_out = jnp.transpose(zq.reshape(4, 8, 16, 16, _E), (0, 4, 1, 2, 3))
    idx = idx3.reshape(_NTOK, 1)
    return (loss[0, 0], z_q_out, ppl[0, 0], oh, idx, z)


# speedup vs baseline: 1.2523x; 1.1370x over previous
import functools

import jax
import jax.numpy as jnp
from jax import lax
from jax.experimental import pallas as pl
from jax.experimental.pallas import tpu as pltpu
from jax.experimental.pallas import tpu_sc as plsc

_N_E = 8192
_E = 32
_BETA = 0.25
_NTOK = 8192
_T = 256
_NT = _NTOK // _T


def _esq_body(emb_ref, esq_ref):
    emb = emb_ref[...]
    esq_ref[...] = jnp.sum(emb * emb, axis=1)[None, :]


def _esq_call(embedding, interpret=False):
    return pl.pallas_call(
        _esq_body,
        out_shape=jax.ShapeDtypeStruct((1, _N_E), jnp.float32),
        interpret=interpret,
    )(embedding)


def _main_body(zf_ref, emb_ref, esq_ref, oh_ref, idx_ref, cnt_ref):
    zf = zf_ref[...]
    emb = emb_ref[...]
    mm = jnp.dot(-2.0 * zf, emb.T, preferred_element_type=jnp.float32)
    zf_sq = jnp.sum(zf * zf, axis=1, keepdims=True)
    d = (zf_sq + esq_ref[...]) + mm
    dmin = jnp.min(d, axis=1, keepdims=True)
    iota1 = jax.lax.broadcasted_iota(jnp.int32, (1, _N_E), 1)
    cand = jnp.where(d == dmin, iota1, _N_E)
    idx = jnp.min(cand, axis=1)
    oh = (cand == idx[:, None]).astype(jnp.float32)
    oh_ref[...] = oh
    idx_ref[...] = idx[None, None, :]
    cnt_ref[...] = jnp.sum(oh, axis=0)[None, None, :]


def _main_call(zf, embedding, esq, interpret=False):
    return pl.pallas_call(
        _main_body,
        grid=(_NT,),
        in_specs=[
            pl.BlockSpec((_T, _E), lambda i: (i, 0)),
            pl.BlockSpec((_N_E, _E), lambda i: (0, 0)),
            pl.BlockSpec((1, _N_E), lambda i: (0, 0)),
        ],
        out_specs=[
            pl.BlockSpec((_T, _N_E), lambda i: (i, 0)),
            pl.BlockSpec((1, 1, _T), lambda i: (i, 0, 0)),
            pl.BlockSpec((1, 1, _N_E), lambda i: (i, 0, 0)),
        ],
        out_shape=[
            jax.ShapeDtypeStruct((_NTOK, _N_E), jnp.float32),
            jax.ShapeDtypeStruct((_NT, 1, _T), jnp.int32),
            jax.ShapeDtypeStruct((_NT, 1, _N_E), jnp.float32),
        ],
        compiler_params=pltpu.CompilerParams(
            dimension_semantics=("parallel",),
        ),
        interpret=interpret,
    )(zf, embedding, esq)


_DPAD = 128


def _gather_call(table_pad, idx_flat):
    info = plsc.get_sparse_core_info()
    nw = info.num_cores * info.num_subcores
    b_per_w = _NTOK // nw
    mesh = plsc.VectorSubcoreMesh(core_axis_name="c", subcore_axis_name="s")

    @functools.partial(
        pl.kernel, mesh=mesh,
        out_type=jax.ShapeDtypeStruct((_NTOK, _DPAD), jnp.float32),
        scratch_types=[
            pltpu.VMEM((b_per_w,), jnp.int32),
            pltpu.VMEM((b_per_w, _DPAD), jnp.float32),
            pltpu.SemaphoreType.DMA,
        ],
    )
    def _k(table_hbm, idx_hbm, out_hbm, idx_v, rows_v, sem):
        wid = lax.axis_index("s") * info.num_cores + lax.axis_index("c")
        base = wid * b_per_w
        pltpu.sync_copy(idx_hbm.at[pl.ds(base, b_per_w)], idx_v)
        pltpu.async_copy(table_hbm.at[idx_v], rows_v, sem).wait()
        pltpu.sync_copy(rows_v, out_hbm.at[pl.ds(base, b_per_w)])

    return _k(table_pad, idx_flat)


def _combine_body(zf_ref, zq_ref, cnt_ref, loss_ref, ppl_ref):
    diff = zq_ref[...] - zf_ref[...]
    part = jnp.sum(diff * diff, axis=(0, 1), keepdims=True)
    m = part * (1.0 / float(_NTOK * _E))
    loss_ref[...] = m + _BETA * m
    cnt = jnp.sum(cnt_ref[...][:, 0, :], axis=0)[None, :]
    e_mean = cnt * (1.0 / _NTOK)
    ent = jnp.sum(e_mean * jnp.log(e_mean + 1e-10), axis=(0, 1),
                  keepdims=True)
    ppl_ref[...] = jnp.exp(-ent)


def _combine_call(zf, zq, cnt3, interpret=False):
    return pl.pallas_call(
        _combine_body,
        out_shape=[
            jax.ShapeDtypeStruct((1, 1), jnp.float32),
            jax.ShapeDtypeStruct((1, 1), jnp.float32),
        ],
        interpret=interpret,
    )(zf, zq, cnt3)


def kernel(z, embedding):
    zp = jnp.transpose(z, (0, 2, 3, 4, 1))
    zf = zp.reshape(_NTOK, _E)
    esq = _esq_call(embedding)
    oh, idx3, cnt3 = _main_call(zf, embedding, esq)
    idx_flat = idx3.reshape(_NTOK)
    emb_pad = jnp.pad(embedding, ((0, 0), (0, _DPAD - _E)))
    zq = _gather_call(emb_pad, idx_flat)[:, :_E]
    loss, ppl = _combine_call(zf, zq, cnt3)
    z_q_out = jnp.transpose(zq.reshape(4, 8, 16, 16, _E), (0, 4, 1, 2, 3))
    idx = idx3.reshape(_NTOK, 1)
    return (loss[0, 0], z_q_out, ppl[0, 0], oh, idx, z)
